# Initial kernel scaffold; baseline (speedup 1.0000x reference)
import jax
import jax.numpy as jnp
from jax.experimental import pallas as pl


def kernel(forward_response_tensor, vocab_size):
    raise NotImplementedError("write your pallas kernel here")


# speedup vs baseline: 2.6262x; 2.6262x over previous
import functools

import jax
import jax.numpy as jnp
from jax import lax
from jax.experimental import pallas as pl
from jax.experimental.pallas import tpu as pltpu
from jax.experimental.pallas import tpu_sc as plsc

_V = 50257
_ROWBUF = 50272
_NW = 32


def _vlog(x):
    bits = plsc.bitcast(x, jnp.int32)
    e = lax.shift_right_logical(bits, 23) - 127
    m = plsc.bitcast(
        jnp.bitwise_or(jnp.bitwise_and(bits, 0x007FFFFF), 0x3F800000),
        jnp.float32,
    )
    big = m > 1.41421356
    m = jnp.where(big, m * 0.5, m)
    e = (e + jnp.where(big, 1, 0)).astype(jnp.float32)
    xr = m - 1.0
    z = xr * xr
    p = jnp.full((16,), 7.0376836292e-2, jnp.float32)
    for c in (-1.1514610310e-1, 1.1676998740e-1, -1.2420140846e-1,
              1.4249322787e-1, -1.6668057665e-1, 2.0000714765e-1,
              -2.4999993993e-1, 3.3333331174e-1):
        p = p * xr + c
    y = xr * z * p
    y = y + e * (-2.12194440e-4)
    y = y - 0.5 * z
    return xr + y + e * 0.693359375


def _decode_body(K, T, in_hbm, out_hbm, in_buf, row_buf, stage_v, stage_i):
    wid = lax.axis_index("s") * 2 + lax.axis_index("c")
    tpw = T // _NW
    groups = K // 16
    iota = lax.iota(jnp.int32, 16)
    g0 = wid * tpw * _V

    def token_body(j, _):
        t = wid * tpw + j
        pltpu.sync_copy(in_hbm.at[pl.ds(t * 2 * K, 2 * K)], in_buf)

        def p1(i, acc):
            vi = i * 32 + 2 * iota
            vals = plsc.load_gather(in_buf, [vi])
            idxf = plsc.load_gather(in_buf, [vi + 1])
            lv = _vlog(vals + 1e-40)
            stage_v[pl.ds(i * 16, 16)] = lv
            stage_i[pl.ds(i * 16, 16)] = idxf.astype(jnp.int32)
            return acc + vals

        acc = lax.fori_loop(0, groups, p1, jnp.zeros((16,), jnp.float32))
        pmass = jnp.sum(acc)
        rem = jnp.clip(1.0 - pmass, 1e-40, 1.0)
        fillv = _vlog(jnp.broadcast_to(rem, (16,)) * (1.0 / (_V - K)))

        tail = row_buf[pl.ds(50256, 16)]

        def fl(i, c):
            row_buf[pl.ds(i * 16, 16)] = fillv
            return c

        lax.fori_loop(0, _ROWBUF // 16, fl, 0)
        row_buf[pl.ds(0, 16)] = jnp.where(iota < j, tail, fillv)

        def sc(i, c):
            v = stage_v[pl.ds(i * 16, 16)]
            ix = stage_i[pl.ds(i * 16, 16)] + j
            plsc.store_scatter(row_buf, [ix], v)
            return c

        lax.fori_loop(0, groups, sc, 0)
        pltpu.sync_copy(row_buf.at[pl.ds(0, 50256)],
                        out_hbm.at[pl.ds(g0 + j * 50256, 50256)])
        return _

    lax.fori_loop(0, tpw, token_body, 0)
    pltpu.sync_copy(row_buf.at[pl.ds(50256, 8)],
                    out_hbm.at[pl.ds(g0 + tpw * _V - 8, 8)])


def kernel(forward_response_tensor, vocab_size):
    del vocab_size
    B, S, K, _two = forward_response_tensor.shape
    T = B * S
    flat = forward_response_tensor.reshape(T * 2 * K)
    mesh = plsc.VectorSubcoreMesh(core_axis_name="c", subcore_axis_name="s")
    f = pl.kernel(
        functools.partial(_decode_body, K, T),
        out_type=jax.ShapeDtypeStruct((T * _V,), jnp.float32),
        mesh=mesh,
        scratch_types=[
            pltpu.VMEM((2 * K,), jnp.float32),
            pltpu.VMEM((_ROWBUF,), jnp.float32),
            pltpu.VMEM((K,), jnp.float32),
            pltpu.VMEM((K,), jnp.int32),
        ],
        compiler_params=pltpu.CompilerParams(needs_layout_passes=False),
    )
    return f(flat).reshape(B, S, _V)


# speedup vs baseline: 37.4555x; 14.2623x over previous
import jax
import jax.numpy as jnp
from jax import lax
from jax.experimental import pallas as pl
from jax.experimental.pallas import tpu as pltpu
from jax.experimental.pallas import tpu_sc as plsc

_V = 50257
_CH = 393
_VPAD = _CH * 128
_NW = 32
_K = 4096
_B = 32
_S = 8


def _vlog(x):
    bits = plsc.bitcast(x, jnp.int32)
    e = lax.shift_right_logical(bits, 23) - 127
    m = plsc.bitcast(
        jnp.bitwise_or(jnp.bitwise_and(bits, 0x007FFFFF), 0x3F800000),
        jnp.float32,
    )
    big = m > 1.41421356
    m = jnp.where(big, m * 0.5, m)
    e = (e + jnp.where(big, 1, 0)).astype(jnp.float32)
    xr = m - 1.0
    z = xr * xr
    p = jnp.full((16,), 7.0376836292e-2, jnp.float32)
    for c in (-1.1514610310e-1, 1.1676998740e-1, -1.2420140846e-1,
              1.4249322787e-1, -1.6668057665e-1, 2.0000714765e-1,
              -2.4999993993e-1, 3.3333331174e-1):
        p = p * xr + c
    y = xr * z * p
    y = y + e * (-2.12194440e-4)
    y = y - 0.5 * z
    return xr + y + e * 0.693359375


def _decode_body(in_hbm, out_hbm, in0, in1, row0, row1, stage_v, stage_i,
                 si0, si1, so0, so1):
    wid = lax.axis_index("s") * 2 + lax.axis_index("c")
    iota = lax.iota(jnp.int32, 16)
    in_bufs = (in0, in1)
    row_bufs = (row0, row1)
    in_sems = (si0, si1)
    out_sems = (so0, so1)

    def start_in(j):
        t = wid * _S + j
        return pltpu.async_copy(
            in_hbm.at[pl.ds(t * 2 * _K, 2 * _K)], in_bufs[j % 2],
            in_sems[j % 2])

    h_in = start_in(0)
    h_out = [None, None]
    for j in range(_S):
        ib = in_bufs[j % 2]
        rb = row_bufs[j % 2]
        h_in.wait()
        if j + 1 < _S:
            h_in = start_in(j + 1)

        def p1(kc, acc):
            base = kc * 256
            parts = []
            for u in range(8):
                v = ib[pl.ds(base + u * 16, 16)]
                ix = ib[pl.ds(base + 128 + u * 16, 16)]
                lv = _vlog(v + 1e-40)
                stage_v[pl.ds(kc * 128 + u * 16, 16)] = lv
                stage_i[pl.ds(kc * 128 + u * 16, 16)] = ix.astype(jnp.int32)
                parts.append(v)
            s01 = (parts[0] + parts[1]) + (parts[2] + parts[3])
            s23 = (parts[4] + parts[5]) + (parts[6] + parts[7])
            return acc + (s01 + s23)

        acc = lax.fori_loop(0, 32, p1, jnp.zeros((16,), jnp.float32))
        pmass = jnp.sum(acc)
        rem = jnp.clip(1.0 - pmass, 1e-40, 1.0)
        fillv = _vlog(jnp.broadcast_to(rem, (16,)) * (1.0 / (_V - _K)))

        if h_out[j % 2] is not None:
            h_out[j % 2].wait()

        def fl(c, carry):
            for u in range(8):
                rb[c, 0, pl.ds(u * 16, 16)] = fillv
            return carry

        lax.fori_loop(0, _CH, fl, 0)

        def sc(i, carry):
            v = stage_v[pl.ds(i * 16, 16)]
            ix = stage_i[pl.ds(i * 16, 16)]
            plsc.store_scatter(
                rb,
                [lax.shift_right_logical(ix, 7),
                 jnp.zeros((16,), jnp.int32),
                 jnp.bitwise_and(ix, 127)],
                v)
            return carry

        lax.fori_loop(0, 256, sc, 0)

        h_out[j % 2] = pltpu.async_copy(
            rb, out_hbm.at[wid, :, pl.ds(j, 1), :], out_sems[j % 2])

    h_out[0].wait()
    h_out[1].wait()


def kernel(forward_response_tensor, vocab_size):
    del vocab_size
    B, S, K, _two = forward_response_tensor.shape
    g = forward_response_tensor.reshape(B, S, K // 128, 128, 2)
    g = g.transpose(0, 1, 2, 4, 3).reshape(B * S * K * 2)
    mesh = plsc.VectorSubcoreMesh(core_axis_name="c", subcore_axis_name="s")
    f = pl.kernel(
        _decode_body,
        out_type=jax.ShapeDtypeStruct((_B, _CH, _S, 128), jnp.float32),
        mesh=mesh,
        scratch_types=[
            pltpu.VMEM((2 * _K,), jnp.float32),
            pltpu.VMEM((2 * _K,), jnp.float32),
            pltpu.VMEM((_CH, 1, 128), jnp.float32),
            pltpu.VMEM((_CH, 1, 128), jnp.float32),
            pltpu.VMEM((_K,), jnp.float32),
            pltpu.VMEM((_K,), jnp.int32),
            pltpu.SemaphoreType.DMA,
            pltpu.SemaphoreType.DMA,
            pltpu.SemaphoreType.DMA,
            pltpu.SemaphoreType.DMA,
        ],
        compiler_params=pltpu.CompilerParams(needs_layout_passes=False),
    )
    o4 = f(g)
    return o4.transpose(0, 2, 1, 3).reshape(_B, _S, _VPAD)[..., :_V]


# speedup vs baseline: 44.6554x; 1.1922x over previous
import jax
import jax.numpy as jnp
from jax import lax
from jax.experimental import pallas as pl
from jax.experimental.pallas import tpu as pltpu
from jax.experimental.pallas import tpu_sc as plsc

_V = 50257
_CH = 393
_VPAD = _CH * 128
_NW = 32
_K = 4096
_B = 32
_S = 8


def _vlog(x):
    bits = plsc.bitcast(x, jnp.int32)
    e = lax.shift_right_logical(bits, 23) - 127
    m = plsc.bitcast(
        jnp.bitwise_or(jnp.bitwise_and(bits, 0x007FFFFF), 0x3F800000),
        jnp.float32,
    )
    big = m > 1.41421356
    m = jnp.where(big, m * 0.5, m)
    e = (e + jnp.where(big, 1, 0)).astype(jnp.float32)
    t = m - 1.0
    p = jnp.full((16,), -0.14166949689388275, jnp.float32)
    for c in (0.21813951432704926, -0.253643274307251, 0.3327617645263672,
              -0.49992313981056213, 1.0000028610229492):
        p = p * t + c
    return t * p + e * 0.6931472


def _decode_body(in_hbm, out_hbm, in0, in1, row0, row1, stage_v, stage_i,
                 si0, si1, so0, so1):
    wid = lax.axis_index("s") * 2 + lax.axis_index("c")
    iota = lax.iota(jnp.int32, 16)
    in_bufs = (in0, in1)
    row_bufs = (row0, row1)
    in_sems = (si0, si1)
    out_sems = (so0, so1)

    def start_in(j):
        t = wid * _S + j
        return pltpu.async_copy(
            in_hbm.at[pl.ds(t * 2 * _K, 2 * _K)], in_bufs[j % 2],
            in_sems[j % 2])

    h_in = start_in(0)
    h_out = [None, None]
    for j in range(_S):
        ib = in_bufs[j % 2]
        rb = row_bufs[j % 2]
        h_in.wait()
        if j + 1 < _S:
            h_in = start_in(j + 1)

        def p1(kc, acc):
            base = kc * 256
            parts = []
            for u in range(8):
                v = ib[pl.ds(base + u * 16, 16)]
                ix = ib[pl.ds(base + 128 + u * 16, 16)]
                lv = _vlog(v + 1e-40)
                stage_v[pl.ds(kc * 128 + u * 16, 16)] = lv
                stage_i[pl.ds(kc * 128 + u * 16, 16)] = ix.astype(jnp.int32)
                parts.append(v)
            s01 = (parts[0] + parts[1]) + (parts[2] + parts[3])
            s23 = (parts[4] + parts[5]) + (parts[6] + parts[7])
            return acc + (s01 + s23)

        acc = lax.fori_loop(0, 32, p1, jnp.zeros((16,), jnp.float32))
        pmass = jnp.sum(acc)
        rem = jnp.clip(1.0 - pmass, 1e-40, 1.0)
        fillv = _vlog(jnp.broadcast_to(rem, (16,)) * (1.0 / (_V - _K)))

        if h_out[j % 2] is not None:
            h_out[j % 2].wait()

        def fl(c, carry):
            for u in range(24):
                rb[c * 3 + u // 8, 0, pl.ds((u % 8) * 16, 16)] = fillv
            return carry

        lax.fori_loop(0, _CH // 3, fl, 0)

        zero16 = jnp.zeros((16,), jnp.int32)

        def sc(i, carry):
            for u in range(4):
                v = stage_v[pl.ds(i * 64 + u * 16, 16)]
                ix = stage_i[pl.ds(i * 64 + u * 16, 16)]
                plsc.store_scatter(
                    rb,
                    [lax.shift_right_logical(ix, 7), zero16,
                     jnp.bitwise_and(ix, 127)],
                    v)
            return carry

        lax.fori_loop(0, 64, sc, 0)

        h_out[j % 2] = pltpu.async_copy(
            rb, out_hbm.at[wid, :, pl.ds(j, 1), :], out_sems[j % 2])

    h_out[0].wait()
    h_out[1].wait()


def kernel(forward_response_tensor, vocab_size):
    del vocab_size
    B, S, K, _two = forward_response_tensor.shape
    g = forward_response_tensor.reshape(B, S, K // 128, 128, 2)
    g = g.transpose(0, 1, 2, 4, 3).reshape(B * S * K * 2)
    mesh = plsc.VectorSubcoreMesh(core_axis_name="c", subcore_axis_name="s")
    f = pl.kernel(
        _decode_body,
        out_type=jax.ShapeDtypeStruct((_B, _CH, _S, 128), jnp.float32),
        mesh=mesh,
        scratch_types=[
            pltpu.VMEM((2 * _K,), jnp.float32),
            pltpu.VMEM((2 * _K,), jnp.float32),
            pltpu.VMEM((_CH, 1, 128), jnp.float32),
            pltpu.VMEM((_CH, 1, 128), jnp.float32),
            pltpu.VMEM((_K,), jnp.float32),
            pltpu.VMEM((_K,), jnp.int32),
            pltpu.SemaphoreType.DMA,
            pltpu.SemaphoreType.DMA,
            pltpu.SemaphoreType.DMA,
            pltpu.SemaphoreType.DMA,
        ],
        compiler_params=pltpu.CompilerParams(needs_layout_passes=False),
    )
    o4 = f(g)
    return o4.transpose(0, 2, 1, 3).reshape(_B, _S, _VPAD)[..., :_V]
